# Initial kernel scaffold; baseline (speedup 1.0000x reference)
import jax
import jax.numpy as jnp
from jax.experimental import pallas as pl


def kernel(x, edge_index, edge_weight, edge_attr, lin1_w, lin2_w, lin2_b, mlp_w0, mlp_b0, mlp_w2, mlp_b2, lin_w, lin_b):
    raise NotImplementedError("write your pallas kernel here")


# speedup vs baseline: 1.3593x; 1.3593x over previous
import functools

import jax
import jax.numpy as jnp
import numpy as np
from jax import lax
from jax.experimental import pallas as pl
from jax.experimental.pallas import tpu as pltpu
from jax.experimental.pallas import tpu_sc as plsc

CUT_UP = 10.0


def _lin1_body(x_ref, w_ref, o_ref):
    o_ref[...] = lax.dot_general(
        x_ref[...], w_ref[...], (((1,), (1,)), ((), ())),
        preferred_element_type=jnp.float32)


def _tc_lin1(x, lin1_w):
    n, h = x.shape
    return pl.pallas_call(
        _lin1_body,
        out_shape=jax.ShapeDtypeStruct((n, h), jnp.float32),
    )(x, lin1_w)


def _filter_body(ea_ref, ew_ref, w0_ref, b0_ref, w2_ref, b2_ref, o_ref):
    ea = ea_ref[...]
    h1 = lax.dot_general(ea, w0_ref[...], (((1,), (1,)), ((), ())),
                         preferred_element_type=jnp.float32) + b0_ref[...]
    h1 = h1 * jax.nn.sigmoid(h1)
    w = lax.dot_general(h1, w2_ref[...], (((1,), (1,)), ((), ())),
                        preferred_element_type=jnp.float32) + b2_ref[...]
    ew = ew_ref[...]
    cut = 0.5 * (jnp.cos(ew * (np.pi / CUT_UP)) + 1.0)
    cut = jnp.where(ew < CUT_UP, cut, 0.0)
    o_ref[...] = w * cut


def _tc_filter(edge_attr, edge_weight, mlp_w0, mlp_b0, mlp_w2, mlp_b2):
    e, nrbf = edge_attr.shape
    nf = mlp_w0.shape[0]
    be = 2000
    grid = e // be
    ew2 = edge_weight.reshape(e, 1)
    b0 = mlp_b0.reshape(1, nf)
    b2 = mlp_b2.reshape(1, nf)
    return pl.pallas_call(
        _filter_body,
        grid=(grid,),
        in_specs=[
            pl.BlockSpec((be, nrbf), lambda i: (i, 0)),
            pl.BlockSpec((be, 1), lambda i: (i, 0)),
            pl.BlockSpec((nf, nrbf), lambda i: (0, 0)),
            pl.BlockSpec((1, nf), lambda i: (0, 0)),
            pl.BlockSpec((nf, nf), lambda i: (0, 0)),
            pl.BlockSpec((1, nf), lambda i: (0, 0)),
        ],
        out_specs=pl.BlockSpec((be, nf), lambda i: (i, 0)),
        out_shape=jax.ShapeDtypeStruct((e, nf), jnp.float32),
    )(edge_attr, ew2, mlp_w0, b0, mlp_w2, b2)


_NC = 2
_NS = 16
_L = 16


def _sc_message_passing(h, w, src, dst):
    n, hd = h.shape
    e = src.shape[0]
    nw = _NC * _NS
    epw = e // nw
    b = 80
    nchunk = epw // b
    nwriters = 10
    rpt = n // nwriters
    rb = 200
    nrb = rpt // rb

    mesh = plsc.VectorSubcoreMesh(core_axis_name="c", subcore_axis_name="s")

    @functools.partial(
        pl.kernel,
        mesh=mesh,
        out_type=jax.ShapeDtypeStruct((_NC, n, hd), jnp.float32),
        scratch_types=[
            pltpu.VMEM((b,), jnp.int32),
            pltpu.VMEM((b,), jnp.int32),
            pltpu.VMEM((b, hd), jnp.float32),
            pltpu.VMEM((b, hd), jnp.float32),
            pltpu.VMEM((rb, hd), jnp.float32),
            pltpu.VMEM_SHARED((n, hd), jnp.float32),
            pltpu.SemaphoreType.DMA,
        ],
    )
    def sc_body(h_hbm, w_hbm, src_hbm, dst_hbm, out_hbm,
                src_v, dst_v, rows_v, w_v, stage_v, agg_sh, sem):
        c = lax.axis_index("c")
        s = lax.axis_index("s")
        wid = s * _NC + c

        zeros = jnp.zeros((_L,), jnp.float32)

        def zero_row(i, _):
            for f in range(hd // _L):
                stage_v[i, pl.ds(f * _L, _L)] = zeros
            return 0

        lax.fori_loop(0, rb, zero_row, 0)

        @pl.when(s < nwriters)
        def _zero():
            for r in range(nrb):
                row0 = s * rpt + r * rb
                pltpu.sync_copy(stage_v, agg_sh.at[pl.ds(row0, rb)])

        plsc.subcore_barrier()

        def chunk_body(j, _):
            base = wid * epw + j * b
            pltpu.sync_copy(src_hbm.at[pl.ds(base, b)], src_v)
            pltpu.sync_copy(dst_hbm.at[pl.ds(base, b)], dst_v)
            pltpu.async_copy(h_hbm.at[src_v], rows_v, sem).wait()
            pltpu.sync_copy(w_hbm.at[pl.ds(base, b)], w_v)

            def mul_body(ei, _2):
                for f in range(hd // _L):
                    sl = pl.ds(f * _L, _L)
                    rows_v[ei, sl] = rows_v[ei, sl] * w_v[ei, sl]
                return 0

            lax.fori_loop(0, b, mul_body, 0)
            pltpu.sync_copy(rows_v, agg_sh.at[dst_v], add=True)
            return 0

        lax.fori_loop(0, nchunk, chunk_body, 0)
        plsc.subcore_barrier()

        @pl.when(s < nwriters)
        def _writeout():
            for r in range(nrb):
                row0 = s * rpt + r * rb
                pltpu.sync_copy(agg_sh.at[pl.ds(row0, rb)], stage_v)
                pltpu.sync_copy(stage_v, out_hbm.at[c].at[pl.ds(row0, rb)])

    return sc_body(h, w, src, dst)


def _tail_body(p_ref, w2_ref, b2_ref, wl_ref, bl_ref, o_ref):
    agg = p_ref[0] + p_ref[1]
    t = lax.dot_general(agg, w2_ref[...], (((1,), (1,)), ((), ())),
                        preferred_element_type=jnp.float32) + b2_ref[...]
    t = t * jax.nn.sigmoid(t)
    o_ref[...] = lax.dot_general(t, wl_ref[...], (((1,), (1,)), ((), ())),
                                 preferred_element_type=jnp.float32) + bl_ref[...]


def _tc_tail(partial, lin2_w, lin2_b, lin_w, lin_b):
    _, n, h = partial.shape
    bn = 2000
    grid = n // bn
    b2 = lin2_b.reshape(1, h)
    bl = lin_b.reshape(1, h)
    return pl.pallas_call(
        _tail_body,
        grid=(grid,),
        in_specs=[
            pl.BlockSpec((_NC, bn, h), lambda i: (0, i, 0)),
            pl.BlockSpec((h, h), lambda i: (0, 0)),
            pl.BlockSpec((1, h), lambda i: (0, 0)),
            pl.BlockSpec((h, h), lambda i: (0, 0)),
            pl.BlockSpec((1, h), lambda i: (0, 0)),
        ],
        out_specs=pl.BlockSpec((bn, h), lambda i: (i, 0)),
        out_shape=jax.ShapeDtypeStruct((n, h), jnp.float32),
    )(partial, lin2_w, b2, lin_w, bl)


def kernel(x, edge_index, edge_weight, edge_attr, lin1_w, lin2_w, lin2_b,
           mlp_w0, mlp_b0, mlp_w2, mlp_b2, lin_w, lin_b):
    src = edge_index[0]
    dst = edge_index[1]
    h = _tc_lin1(x, lin1_w)
    w = _tc_filter(edge_attr, edge_weight, mlp_w0, mlp_b0, mlp_w2, mlp_b2)
    partial = _sc_message_passing(h, w, src, dst)
    return _tc_tail(partial, lin2_w, lin2_b, lin_w, lin_b)


# speedup vs baseline: 3.3259x; 2.4469x over previous
import functools

import jax
import jax.numpy as jnp
import numpy as np
from jax import lax
from jax.experimental import pallas as pl
from jax.experimental.pallas import tpu as pltpu
from jax.experimental.pallas import tpu_sc as plsc

CUT_UP = 10.0


def _lin1_body(x_ref, w_ref, o_ref):
    o_ref[...] = lax.dot_general(
        x_ref[...], w_ref[...], (((1,), (1,)), ((), ())),
        preferred_element_type=jnp.float32)


def _tc_lin1(x, lin1_w):
    n, h = x.shape
    return pl.pallas_call(
        _lin1_body,
        out_shape=jax.ShapeDtypeStruct((n, h), jnp.float32),
    )(x, lin1_w)


def _filter_body(ea_ref, ew_ref, w0_ref, b0_ref, w2_ref, b2_ref, o_ref):
    ea = ea_ref[...]
    h1 = lax.dot_general(ea, w0_ref[...], (((1,), (1,)), ((), ())),
                         preferred_element_type=jnp.float32) + b0_ref[...]
    h1 = h1 * jax.nn.sigmoid(h1)
    w = lax.dot_general(h1, w2_ref[...], (((1,), (1,)), ((), ())),
                        preferred_element_type=jnp.float32) + b2_ref[...]
    ew = ew_ref[0]
    cut = 0.5 * (jnp.cos(ew * (np.pi / CUT_UP)) + 1.0)
    cut = jnp.where(ew < CUT_UP, cut, 0.0)
    o_ref[...] = w * jnp.transpose(cut, (1, 0))


def _tc_filter(edge_attr, edge_weight, mlp_w0, mlp_b0, mlp_w2, mlp_b2):
    e, nrbf = edge_attr.shape
    nf = mlp_w0.shape[0]
    be = 2000
    grid = e // be
    ew2 = edge_weight.reshape(grid, 1, be)
    b0 = mlp_b0.reshape(1, nf)
    b2 = mlp_b2.reshape(1, nf)
    return pl.pallas_call(
        _filter_body,
        grid=(grid,),
        in_specs=[
            pl.BlockSpec((be, nrbf), lambda i: (i, 0)),
            pl.BlockSpec((1, 1, be), lambda i: (i, 0, 0)),
            pl.BlockSpec((nf, nrbf), lambda i: (0, 0)),
            pl.BlockSpec((1, nf), lambda i: (0, 0)),
            pl.BlockSpec((nf, nf), lambda i: (0, 0)),
            pl.BlockSpec((1, nf), lambda i: (0, 0)),
        ],
        out_specs=pl.BlockSpec((be, nf), lambda i: (i, 0)),
        out_shape=jax.ShapeDtypeStruct((e, nf), jnp.float32),
    )(edge_attr, ew2, mlp_w0, b0, mlp_w2, b2)


_NC = 2
_NS = 16
_L = 16


def _sc_message_passing(h, w, src, dst):
    n, hd = h.shape
    e = src.shape[0]
    nw = _NC * _NS
    epw = e // nw
    b = 80
    nchunk = epw // b
    nrows_chunks = -(-n // b)
    rounds = -(-nrows_chunks // _NS)

    mesh = plsc.VectorSubcoreMesh(core_axis_name="c", subcore_axis_name="s")

    @functools.partial(
        pl.kernel,
        mesh=mesh,
        out_type=jax.ShapeDtypeStruct((_NC, n, hd), jnp.float32),
        scratch_types=[
            pltpu.VMEM((b,), jnp.int32),
            pltpu.VMEM((b,), jnp.int32),
            pltpu.VMEM((b,), jnp.int32),
            pltpu.VMEM((b,), jnp.int32),
            pltpu.VMEM((b, hd), jnp.float32),
            pltpu.VMEM((b, hd), jnp.float32),
            pltpu.VMEM((b, hd), jnp.float32),
            pltpu.VMEM((b, hd), jnp.float32),
            pltpu.VMEM_SHARED((n, hd), jnp.float32),
            pltpu.SemaphoreType.DMA,
            pltpu.SemaphoreType.DMA,
            pltpu.SemaphoreType.DMA,
            pltpu.SemaphoreType.DMA,
            pltpu.SemaphoreType.DMA,
            pltpu.SemaphoreType.DMA,
        ],
    )
    def sc_body(h_hbm, w_hbm, src_hbm, dst_hbm, out_hbm,
                src0, src1, dst0, dst1, rows0, rows1, w0, w1,
                agg_sh, isem0, isem1, gsem0, gsem1, wsem0, wsem1):
        c = lax.axis_index("c")
        s = lax.axis_index("s")
        wid = s * _NC + c

        srcs = (src0, src1)
        dsts = (dst0, dst1)
        rows = (rows0, rows1)
        ws = (w0, w1)
        isems = (isem0, isem1)
        gsems = (gsem0, gsem1)
        wsems = (wsem0, wsem1)

        zeros = jnp.zeros((_L,), jnp.float32)

        def zero_row(i, _):
            for f in range(hd // _L):
                rows0[i, pl.ds(f * _L, _L)] = zeros
            return 0

        lax.fori_loop(0, b, zero_row, 0)

        def zero_chunk(k, _):
            idx = s + k * _NS

            @pl.when(idx < nrows_chunks)
            def _z():
                pltpu.sync_copy(rows0, agg_sh.at[pl.ds(idx * b, b)])

            return 0

        lax.fori_loop(0, rounds, zero_chunk, 0)
        plsc.subcore_barrier()

        def idx_start(j, sl):
            base = wid * epw + j * b
            pltpu.async_copy(src_hbm.at[pl.ds(base, b)], srcs[sl], isems[sl])
            pltpu.async_copy(dst_hbm.at[pl.ds(base, b)], dsts[sl], isems[sl])

        def idx_wait(sl):
            pltpu.make_async_copy(src_hbm.at[pl.ds(0, b)], srcs[sl], isems[sl]).wait()
            pltpu.make_async_copy(dst_hbm.at[pl.ds(0, b)], dsts[sl], isems[sl]).wait()

        def fetch_start(j, sl):
            base = wid * epw + j * b
            pltpu.async_copy(h_hbm.at[srcs[sl]], rows[sl], gsems[sl])
            pltpu.async_copy(w_hbm.at[pl.ds(base, b)], ws[sl], wsems[sl])

        def process(sl):
            pltpu.make_async_copy(h_hbm.at[srcs[sl]], rows[sl], gsems[sl]).wait()
            pltpu.make_async_copy(w_hbm.at[pl.ds(0, b)], ws[sl], wsems[sl]).wait()
            rv = rows[sl]
            wv = ws[sl]

            def mul_body(k, _2):
                for u in range(2):
                    for f in range(hd // _L):
                        sl2 = pl.ds(f * _L, _L)
                        ei = k * 2 + u
                        rv[ei, sl2] = rv[ei, sl2] * wv[ei, sl2]
                return 0

            lax.fori_loop(0, b // 2, mul_body, 0)
            pltpu.sync_copy(rv, agg_sh.at[dsts[sl]], add=True)

        idx_start(0, 0)
        idx_wait(0)
        fetch_start(0, 0)
        idx_start(1, 1)

        def step(j, sl):
            other = 1 - sl

            @pl.when(j + 1 < nchunk)
            def _g():
                idx_wait(other)
                fetch_start(j + 1, other)

            process(sl)

            @pl.when(j + 2 < nchunk)
            def _i():
                idx_start(j + 2, sl)

        def pair(k, _):
            step(k * 2, 0)
            step(k * 2 + 1, 1)
            return 0

        lax.fori_loop(0, nchunk // 2, pair, 0)
        if nchunk % 2 == 1:
            step(nchunk - 1, 0)
        plsc.subcore_barrier()

        def out_chunk(k, _):
            idx = s + k * _NS

            @pl.when(idx < nrows_chunks)
            def _o():
                pltpu.sync_copy(agg_sh.at[pl.ds(idx * b, b)], rows0)
                pltpu.sync_copy(rows0, out_hbm.at[c].at[pl.ds(idx * b, b)])

            return 0

        lax.fori_loop(0, rounds, out_chunk, 0)

    return sc_body(h, w, src, dst)


def _tail_body(p_ref, w2_ref, b2_ref, wl_ref, bl_ref, o_ref):
    agg = p_ref[0] + p_ref[1]
    t = lax.dot_general(agg, w2_ref[...], (((1,), (1,)), ((), ())),
                        preferred_element_type=jnp.float32) + b2_ref[...]
    t = t * jax.nn.sigmoid(t)
    o_ref[...] = lax.dot_general(t, wl_ref[...], (((1,), (1,)), ((), ())),
                                 preferred_element_type=jnp.float32) + bl_ref[...]


def _tc_tail(partial, lin2_w, lin2_b, lin_w, lin_b):
    _, n, h = partial.shape
    bn = 2000
    grid = n // bn
    b2 = lin2_b.reshape(1, h)
    bl = lin_b.reshape(1, h)
    return pl.pallas_call(
        _tail_body,
        grid=(grid,),
        in_specs=[
            pl.BlockSpec((_NC, bn, h), lambda i: (0, i, 0)),
            pl.BlockSpec((h, h), lambda i: (0, 0)),
            pl.BlockSpec((1, h), lambda i: (0, 0)),
            pl.BlockSpec((h, h), lambda i: (0, 0)),
            pl.BlockSpec((1, h), lambda i: (0, 0)),
        ],
        out_specs=pl.BlockSpec((bn, h), lambda i: (i, 0)),
        out_shape=jax.ShapeDtypeStruct((n, h), jnp.float32),
    )(partial, lin2_w, b2, lin_w, bl)


def kernel(x, edge_index, edge_weight, edge_attr, lin1_w, lin2_w, lin2_b,
           mlp_w0, mlp_b0, mlp_w2, mlp_b2, lin_w, lin_b):
    src = edge_index[0]
    dst = edge_index[1]
    h = _tc_lin1(x, lin1_w)
    w = _tc_filter(edge_attr, edge_weight, mlp_w0, mlp_b0, mlp_w2, mlp_b2)
    partial = _sc_message_passing(h, w, src, dst)
    return _tc_tail(partial, lin2_w, lin2_b, lin_w, lin_b)
